# baseline (speedup 1.0000x reference)
import jax
import jax.numpy as jnp
from jax import lax
from jax.experimental import pallas as pl
from jax.experimental.pallas import tpu as pltpu
from jax.experimental.pallas import tpu_sc as plsc

NHEAD = 16
NBUCKET = 5
NC = 2
NS = 16
LANES = 16
NW = NC * NS

BR = 8
BC = 256


def _make_sc_call(l):
  rows_per_w = l // NW
  nslabs = rows_per_w // BR
  nchunks = l // BC
  nblocks = nslabs * nchunks
  assert nblocks % 2 == 0
  mesh = plsc.VectorSubcoreMesh(core_axis_name="c", subcore_axis_name="s")

  def body(bucket_hbm, table_hbm, out_hbm, table_v,
           bucket0, bucket1, out0, out1,
           in_sem0, in_sem1, out_sem0, out_sem1):
    wid = lax.axis_index("s") * NC + lax.axis_index("c")
    row0 = wid * rows_per_w
    pltpu.sync_copy(table_hbm, table_v)

    buckets = (bucket0, bucket1)
    outs = (out0, out1)
    in_sems = (in_sem0, in_sem1)
    out_sems = (out_sem0, out_sem1)

    def rowslice(blk):
      return pl.ds(row0 + (blk // nchunks) * BR, BR)

    def colslice(blk):
      return pl.ds((blk % nchunks) * BC, BC)

    def in_src(blk):
      return bucket_hbm.at[rowslice(blk), colslice(blk)]

    def out_dst(blk):
      return out_hbm.at[:, rowslice(blk), colslice(blk)]

    for s in range(2):
      pltpu.async_copy(in_src(s), buckets[s], in_sems[s])

    def blk2_body(i0, carry):
      for s in range(2):
        blk = i0 * 2 + s
        bucket_v, out_v = buckets[s], outs[s]
        pltpu.make_async_copy(in_src(blk), bucket_v, in_sems[s]).wait()

        @pl.when(blk >= 2)
        def _():
          pltpu.make_async_copy(out_v, out_dst(blk - 2), out_sems[s]).wait()

        @plsc.parallel_loop(0, BR * BC // LANES, step=1)
        def vec_body(v):
          r = v >> 4
          c = (v & (BC // LANES - 1)) * LANES
          bvec = bucket_v[r, pl.ds(c, LANES)]
          for h in range(NHEAD):
            idx = bvec + (h * NBUCKET)
            out_v[h, r, pl.ds(c, LANES)] = plsc.load_gather(table_v, [idx])

        @pl.when(blk + 2 < nblocks)
        def _():
          pltpu.async_copy(in_src(blk + 2), bucket_v, in_sems[s])

        pltpu.async_copy(out_v, out_dst(blk), out_sems[s])
      return carry

    lax.fori_loop(0, nblocks // 2, blk2_body, 0)
    for s in range(2):
      pltpu.make_async_copy(outs[s], out_dst(nblocks - 2 + s), out_sems[s]).wait()

  return pl.kernel(
      body,
      out_type=jax.ShapeDtypeStruct((NHEAD, l, l), jnp.float32),
      mesh=mesh,
      scratch_types=[
          pltpu.VMEM((128,), jnp.float32),
          pltpu.VMEM((BR, BC), jnp.int32),
          pltpu.VMEM((BR, BC), jnp.int32),
          pltpu.VMEM((NHEAD, BR, BC), jnp.float32),
          pltpu.VMEM((NHEAD, BR, BC), jnp.float32),
          pltpu.SemaphoreType.DMA,
          pltpu.SemaphoreType.DMA,
          pltpu.SemaphoreType.DMA,
          pltpu.SemaphoreType.DMA,
      ],
      compiler_params=pltpu.CompilerParams(
          needs_layout_passes=False, use_tc_tiling_on_sc=True),
  )


@jax.jit
def kernel(bucket_matrix, bias_table):
  b, l, l2 = bucket_matrix.shape
  bm = bucket_matrix.astype(jnp.int32).reshape(l, l2)
  table = jnp.pad(bias_table.astype(jnp.float32).reshape(NHEAD * NBUCKET),
                  (0, 128 - NHEAD * NBUCKET))
  out = _make_sc_call(l)(bm, table)
  return out.reshape(b, NHEAD, l, l2)
